# Initial kernel scaffold; baseline (speedup 1.0000x reference)
import jax
import jax.numpy as jnp
from jax.experimental import pallas as pl


def kernel(x, edge_index, edge_attr, W1, b1, W2, b2):
    raise NotImplementedError("write your pallas kernel here")


# speedup vs baseline: 10.2625x; 10.2625x over previous
import functools

import jax
import jax.numpy as jnp
from jax import lax
from jax.experimental import pallas as pl
from jax.experimental.pallas import tpu as pltpu
from jax.experimental.pallas import tpu_sc as plsc

N = 10000
E = 320000
D = 128
H = 16
C = 40

NP = 10240
EP = 327680
NQ = 8
F = 4
EPQ = EP // NQ
CHUNK = 8192


def _sc_aggregate(table_flat, src, dst, w):
    mesh = plsc.VectorSubcoreMesh(core_axis_name="c", subcore_axis_name="s")

    @functools.partial(
        pl.kernel,
        mesh=mesh,
        compiler_params=pltpu.CompilerParams(needs_layout_passes=False),
        out_type=jax.ShapeDtypeStruct((NQ * H * NP,), jnp.float32),
        scratch_types=[
            pltpu.VMEM((F * NP,), jnp.float32),
            pltpu.VMEM((F * NP,), jnp.float32),
            pltpu.VMEM((CHUNK,), jnp.int32),
            pltpu.VMEM((CHUNK,), jnp.int32),
            pltpu.VMEM((CHUNK,), jnp.float32),
        ],
    )
    def agg(table_hbm, src_hbm, dst_hbm, w_hbm, out_hbm,
            table_v, acc_v, src_v, dst_v, w_v):
        c = lax.axis_index("c")
        s = lax.axis_index("s")
        g = s // 8
        q = s % 8
        f0 = 8 * c + F * g
        e0 = q * EPQ

        for j in range(F):
            pltpu.sync_copy(table_hbm.at[pl.ds((f0 + j) * NP, NP)],
                            table_v.at[pl.ds(j * NP, NP)])

        def _zero(i, _):
            off = pl.multiple_of(i * 16, 16)
            acc_v[pl.ds(off, 16)] = jnp.zeros((16,), jnp.float32)
            return 0
        lax.fori_loop(0, (F * NP) // 16, _zero, 0)

        for ci in range(EPQ // CHUNK):
            eoff = e0 + ci * CHUNK
            pltpu.sync_copy(src_hbm.at[pl.ds(eoff, CHUNK)], src_v)
            pltpu.sync_copy(dst_hbm.at[pl.ds(eoff, CHUNK)], dst_v)
            pltpu.sync_copy(w_hbm.at[pl.ds(eoff, CHUNK)], w_v)

            def _edges(i, _):
                base = pl.multiple_of(i * 16, 16)
                srcs = src_v[pl.ds(base, 16)]
                dsts = dst_v[pl.ds(base, 16)]
                wv = w_v[pl.ds(base, 16)]
                for j in range(F):
                    vals = plsc.load_gather(table_v, [srcs + (j * NP)])
                    plsc.addupdate_scatter(acc_v, [dsts + (j * NP)],
                                           vals * wv)
                return 0
            lax.fori_loop(0, CHUNK // 16, _edges, 0)

        for j in range(F):
            pltpu.sync_copy(acc_v.at[pl.ds(j * NP, NP)],
                            out_hbm.at[pl.ds((q * H + f0 + j) * NP, NP)])

    return agg(table_flat, src, dst, w)


def _project_kernel(x_ref, w1_ref, out_ref):
    out_ref[...] = lax.dot_general(
        w1_ref[...], x_ref[...], (((0,), (1,)), ((), ())),
        preferred_element_type=jnp.float32)


def _combine_relu_kernel(p_ref, b1_ref, out_ref):
    s = p_ref[0:H, :]
    for q in range(1, NQ):
        s = s + p_ref[q * H:(q + 1) * H, :]
    out_ref[...] = jnp.maximum(s + b1_ref[...], 0.0)


def _final_kernel(p_ref, w2_ref, b2_ref, out_ref):
    s = p_ref[0:H, 0:N]
    for q in range(1, NQ):
        s = s + p_ref[q * H:(q + 1) * H, 0:N]
    z = lax.dot_general(s, w2_ref[...], (((0,), (0,)), ((), ())),
                        preferred_element_type=jnp.float32)
    z = z + b2_ref[...]
    m = jnp.max(z, axis=1, keepdims=True)
    zm = z - m
    lse = jnp.log(jnp.sum(jnp.exp(zm), axis=1, keepdims=True))
    out_ref[...] = zm - lse


def kernel(x, edge_index, edge_attr, W1, b1, W2, b2):
    src = jnp.pad(edge_index[0], (0, EP - E))
    dst = jnp.pad(edge_index[1], (0, EP - E))
    w = jnp.pad(edge_attr, (0, EP - E))
    x_pad = jnp.pad(x, ((0, NP - N), (0, 0)))

    h1_t = pl.pallas_call(
        _project_kernel,
        out_shape=jax.ShapeDtypeStruct((H, NP), jnp.float32),
    )(x_pad, W1)

    p1 = _sc_aggregate(h1_t.reshape(H * NP), src, dst, w)

    r_t = pl.pallas_call(
        _combine_relu_kernel,
        out_shape=jax.ShapeDtypeStruct((H, NP), jnp.float32),
    )(p1.reshape(NQ * H, NP), b1.reshape(H, 1))

    p2 = _sc_aggregate(r_t.reshape(H * NP), src, dst, w)

    out = pl.pallas_call(
        _final_kernel,
        out_shape=jax.ShapeDtypeStruct((N, C), jnp.float32),
    )(p2.reshape(NQ * H, NP), W2, b2)

    return out


# speedup vs baseline: 15.2774x; 1.4887x over previous
import functools

import jax
import jax.numpy as jnp
from jax import lax
from jax.experimental import pallas as pl
from jax.experimental.pallas import tpu as pltpu
from jax.experimental.pallas import tpu_sc as plsc

N = 10000
E = 320000
D = 128
H = 16
C = 40

NP = 10240
EP = 327680
NQ = 8
F = 4
EPQ = EP // NQ
CHUNK = 8192


def _sc_aggregate(table_flat, src, dst, w):
    mesh = plsc.VectorSubcoreMesh(core_axis_name="c", subcore_axis_name="s")

    @functools.partial(
        pl.kernel,
        mesh=mesh,
        compiler_params=pltpu.CompilerParams(needs_layout_passes=False),
        out_type=jax.ShapeDtypeStruct((NQ * H * NP,), jnp.float32),
        scratch_types=[
            pltpu.VMEM((F * NP,), jnp.float32),
            pltpu.VMEM((F * NP,), jnp.float32),
            pltpu.VMEM((CHUNK,), jnp.int32),
            pltpu.VMEM((CHUNK,), jnp.int32),
            pltpu.VMEM((CHUNK,), jnp.float32),
        ],
    )
    def agg(table_hbm, src_hbm, dst_hbm, w_hbm, out_hbm,
            table_v, acc_v, src_v, dst_v, w_v):
        c = lax.axis_index("c")
        s = lax.axis_index("s")
        g = s // 8
        q = s % 8
        f0 = 8 * c + F * g
        e0 = q * EPQ

        for j in range(F):
            pltpu.sync_copy(table_hbm.at[pl.ds((f0 + j) * NP, NP)],
                            table_v.at[pl.ds(j * NP, NP)])

        @plsc.parallel_loop(0, F * NP, 16, unroll=8)
        def _zero(i):
            off = pl.multiple_of(i, 16)
            acc_v[pl.ds(off, 16)] = jnp.zeros((16,), jnp.float32)

        for ci in range(EPQ // CHUNK):
            eoff = e0 + ci * CHUNK
            pltpu.sync_copy(src_hbm.at[pl.ds(eoff, CHUNK)], src_v)
            pltpu.sync_copy(dst_hbm.at[pl.ds(eoff, CHUNK)], dst_v)
            pltpu.sync_copy(w_hbm.at[pl.ds(eoff, CHUNK)], w_v)

            @plsc.parallel_loop(0, CHUNK, 16, unroll=4)
            def _edges(i):
                base = pl.multiple_of(i, 16)
                srcs = src_v[pl.ds(base, 16)]
                dsts = dst_v[pl.ds(base, 16)]
                wv = w_v[pl.ds(base, 16)]
                for j in range(F):
                    vals = plsc.load_gather(table_v, [srcs + (j * NP)])
                    plsc.addupdate_scatter(acc_v, [dsts + (j * NP)],
                                           vals * wv)

        for j in range(F):
            pltpu.sync_copy(acc_v.at[pl.ds(j * NP, NP)],
                            out_hbm.at[pl.ds((q * H + f0 + j) * NP, NP)])

    return agg(table_flat, src, dst, w)


def _project_kernel(x_ref, w1_ref, out_ref):
    out_ref[...] = lax.dot_general(
        w1_ref[...], x_ref[...], (((0,), (1,)), ((), ())),
        preferred_element_type=jnp.float32)


def _combine_relu_kernel(p_ref, b1_ref, out_ref):
    s = p_ref[0:H, :]
    for q in range(1, NQ):
        s = s + p_ref[q * H:(q + 1) * H, :]
    out_ref[...] = jnp.maximum(s + b1_ref[...], 0.0)


def _final_kernel(p_ref, w2_ref, b2_ref, out_ref):
    s = p_ref[0:H, 0:N]
    for q in range(1, NQ):
        s = s + p_ref[q * H:(q + 1) * H, 0:N]
    z = lax.dot_general(s, w2_ref[...], (((0,), (0,)), ((), ())),
                        preferred_element_type=jnp.float32)
    z = z + b2_ref[...]
    m = jnp.max(z, axis=1, keepdims=True)
    zm = z - m
    lse = jnp.log(jnp.sum(jnp.exp(zm), axis=1, keepdims=True))
    out_ref[...] = zm - lse


def kernel(x, edge_index, edge_attr, W1, b1, W2, b2):
    src = jnp.pad(edge_index[0], (0, EP - E))
    dst = jnp.pad(edge_index[1], (0, EP - E))
    w = jnp.pad(edge_attr, (0, EP - E))
    x_pad = jnp.pad(x, ((0, NP - N), (0, 0)))

    h1_t = pl.pallas_call(
        _project_kernel,
        out_shape=jax.ShapeDtypeStruct((H, NP), jnp.float32),
    )(x_pad, W1)

    p1 = _sc_aggregate(h1_t.reshape(H * NP), src, dst, w)

    r_t = pl.pallas_call(
        _combine_relu_kernel,
        out_shape=jax.ShapeDtypeStruct((H, NP), jnp.float32),
    )(p1.reshape(NQ * H, NP), b1.reshape(H, 1))

    p2 = _sc_aggregate(r_t.reshape(H * NP), src, dst, w)

    out = pl.pallas_call(
        _final_kernel,
        out_shape=jax.ShapeDtypeStruct((N, C), jnp.float32),
    )(p2.reshape(NQ * H, NP), W2, b2)

    return out


# speedup vs baseline: 25.2353x; 1.6518x over previous
import functools

import jax
import jax.numpy as jnp
from jax import lax
from jax.experimental import pallas as pl
from jax.experimental.pallas import tpu as pltpu
from jax.experimental.pallas import tpu_sc as plsc

N = 10000
E = 320000
D = 128
H = 16
C = 40

NQ = 8
F = 4
EPQ = E // NQ
CHUNK = 4000
NCHUNK = EPQ // CHUNK


def _sc_aggregate(table_flat, src, dst, w):
    mesh = plsc.VectorSubcoreMesh(core_axis_name="c", subcore_axis_name="s")

    @functools.partial(
        pl.kernel,
        mesh=mesh,
        compiler_params=pltpu.CompilerParams(needs_layout_passes=False),
        out_type=jax.ShapeDtypeStruct((NQ * H * N,), jnp.float32),
        scratch_types=[
            pltpu.VMEM((F * N,), jnp.float32),
            pltpu.VMEM((F * N,), jnp.float32),
            pltpu.VMEM((CHUNK,), jnp.int32),
            pltpu.VMEM((CHUNK,), jnp.int32),
            pltpu.VMEM((CHUNK,), jnp.int32),
            pltpu.VMEM((CHUNK,), jnp.int32),
            pltpu.VMEM((CHUNK,), jnp.float32),
            pltpu.VMEM((CHUNK,), jnp.float32),
            pltpu.SemaphoreType.DMA,
            pltpu.SemaphoreType.DMA,
            pltpu.SemaphoreType.DMA,
        ],
    )
    def agg(table_hbm, src_hbm, dst_hbm, w_hbm, out_hbm,
            table_v, acc_v, src_v0, src_v1, dst_v0, dst_v1, w_v0, w_v1,
            sem_t, sem_c0, sem_c1):
        c = lax.axis_index("c")
        s = lax.axis_index("s")
        g = s // 8
        q = s % 8
        f0 = 8 * c + F * g
        e0 = q * EPQ
        sems = [sem_c0, sem_c1]
        src_bufs = [src_v0, src_v1]
        dst_bufs = [dst_v0, dst_v1]
        w_bufs = [w_v0, w_v1]

        tcopies = [
            pltpu.async_copy(table_hbm.at[pl.ds((f0 + j) * N, N)],
                             table_v.at[pl.ds(j * N, N)], sem_t)
            for j in range(F)
        ]

        def _fire(ci):
            buf = ci % 2
            eoff = e0 + ci * CHUNK
            sem = sems[buf]
            return [
                pltpu.async_copy(src_hbm.at[pl.ds(eoff, CHUNK)],
                                 src_bufs[buf], sem),
                pltpu.async_copy(dst_hbm.at[pl.ds(eoff, CHUNK)],
                                 dst_bufs[buf], sem),
                pltpu.async_copy(w_hbm.at[pl.ds(eoff, CHUNK)],
                                 w_bufs[buf], sem),
            ]

        inflight = _fire(0)

        @plsc.parallel_loop(0, F * N, 16, unroll=8)
        def _zero(i):
            off = pl.multiple_of(i, 16)
            acc_v[pl.ds(off, 16)] = jnp.zeros((16,), jnp.float32)

        for t in tcopies:
            t.wait()

        for ci in range(NCHUNK):
            buf = ci % 2
            for h in inflight:
                h.wait()
            if ci + 1 < NCHUNK:
                inflight = _fire(ci + 1)

            src_v, dst_v, w_v = src_bufs[buf], dst_bufs[buf], w_bufs[buf]

            @plsc.parallel_loop(0, CHUNK, 16, unroll=8)
            def _edges(i):
                base = pl.multiple_of(i, 16)
                srcs = src_v[pl.ds(base, 16)]
                dsts = dst_v[pl.ds(base, 16)]
                wv = w_v[pl.ds(base, 16)]
                for j in range(F):
                    vals = plsc.load_gather(table_v, [srcs + (j * N)])
                    plsc.addupdate_scatter(acc_v, [dsts + (j * N)],
                                           vals * wv)

        for j in range(F):
            pltpu.sync_copy(acc_v.at[pl.ds(j * N, N)],
                            out_hbm.at[pl.ds((q * H + f0 + j) * N, N)])

    return agg(table_flat, src, dst, w)


def _project_kernel(x_ref, w1_ref, out_ref):
    out_ref[...] = lax.dot_general(
        w1_ref[...], x_ref[...], (((0,), (1,)), ((), ())),
        preferred_element_type=jnp.float32)


def _combine_relu_kernel(p_ref, b1_ref, out_ref):
    s = p_ref[0:H, :]
    for q in range(1, NQ):
        s = s + p_ref[q * H:(q + 1) * H, :]
    out_ref[...] = jnp.maximum(s + b1_ref[...], 0.0)


def _final_kernel(p_ref, w2_ref, b2_ref, out_ref):
    s = p_ref[0:H, :]
    for q in range(1, NQ):
        s = s + p_ref[q * H:(q + 1) * H, :]
    z = lax.dot_general(s, w2_ref[...], (((0,), (0,)), ((), ())),
                        preferred_element_type=jnp.float32)
    z = z + b2_ref[...]
    m = jnp.max(z, axis=1, keepdims=True)
    zm = z - m
    lse = jnp.log(jnp.sum(jnp.exp(zm), axis=1, keepdims=True))
    out_ref[...] = zm - lse


def kernel(x, edge_index, edge_attr, W1, b1, W2, b2):
    src = edge_index[0]
    dst = edge_index[1]

    h1_t = pl.pallas_call(
        _project_kernel,
        out_shape=jax.ShapeDtypeStruct((H, N), jnp.float32),
    )(x, W1)

    p1 = _sc_aggregate(h1_t.reshape(H * N), src, dst, edge_attr)

    r_t = pl.pallas_call(
        _combine_relu_kernel,
        out_shape=jax.ShapeDtypeStruct((H, N), jnp.float32),
    )(p1.reshape(NQ * H, N), b1.reshape(H, 1))

    p2 = _sc_aggregate(r_t.reshape(H * N), src, dst, edge_attr)

    out = pl.pallas_call(
        _final_kernel,
        out_shape=jax.ShapeDtypeStruct((N, C), jnp.float32),
    )(p2.reshape(NQ * H, N), W2, b2)

    return out


# speedup vs baseline: 28.6343x; 1.1347x over previous
import functools

import jax
import jax.numpy as jnp
from jax import lax
from jax.experimental import pallas as pl
from jax.experimental.pallas import tpu as pltpu
from jax.experimental.pallas import tpu_sc as plsc

N = 10000
E = 320000
D = 128
H = 16
C = 40

NQ = 8
F = 4
EPQ = E // NQ
CHUNK = 4000
NCHUNK = EPQ // CHUNK
SL = 1264
SLSTEP = 1248


def _sc_gcn(table_flat, edge_index, w, b1):
    mesh = plsc.VectorSubcoreMesh(core_axis_name="c", subcore_axis_name="s")

    @functools.partial(
        pl.kernel,
        mesh=mesh,
        compiler_params=pltpu.CompilerParams(needs_layout_passes=False),
        out_type=[
            jax.ShapeDtypeStruct((NQ * H * N,), jnp.float32),
            jax.ShapeDtypeStruct((NQ * H * N,), jnp.float32),
            jax.ShapeDtypeStruct((H * N,), jnp.float32),
        ],
        scratch_types=[
            pltpu.VMEM((F * N,), jnp.float32),
            pltpu.VMEM((F * N,), jnp.float32),
            pltpu.VMEM((CHUNK,), jnp.int32),
            pltpu.VMEM((CHUNK,), jnp.int32),
            pltpu.VMEM((CHUNK,), jnp.int32),
            pltpu.VMEM((CHUNK,), jnp.int32),
            pltpu.VMEM((CHUNK,), jnp.float32),
            pltpu.VMEM((CHUNK,), jnp.float32),
            pltpu.VMEM((NQ * SL,), jnp.float32),
            pltpu.VMEM((SL,), jnp.float32),
            pltpu.VMEM((16,), jnp.float32),
            pltpu.SemaphoreType.DMA,
            pltpu.SemaphoreType.DMA,
            pltpu.SemaphoreType.DMA,
        ],
    )
    def gcn(table_hbm, edge_hbm, w_hbm, b1_hbm, p2_hbm, stage_hbm, rt_hbm,
            table_v, acc_v, src_v0, src_v1, dst_v0, dst_v1, w_v0, w_v1,
            red_v, comb_v, b1_v, sem_t, sem_c0, sem_c1):
        c = lax.axis_index("c")
        s = lax.axis_index("s")
        g = s // 8
        q = s % 8
        f0 = 8 * c + F * g
        e0 = q * EPQ
        sems = [sem_c0, sem_c1]
        src_bufs = [src_v0, src_v1]
        dst_bufs = [dst_v0, dst_v1]
        w_bufs = [w_v0, w_v1]

        def _load_table(src_hbm_flat):
            return [
                pltpu.async_copy(src_hbm_flat.at[pl.ds((f0 + j) * N, N)],
                                 table_v.at[pl.ds(j * N, N)], sem_t)
                for j in range(F)
            ]

        def _zero_acc():
            @plsc.parallel_loop(0, F * N, 16, unroll=8)
            def _zero(i):
                off = pl.multiple_of(i, 16)
                acc_v[pl.ds(off, 16)] = jnp.zeros((16,), jnp.float32)

        def _fire(ci):
            buf = ci % 2
            eoff = e0 + ci * CHUNK
            sem = sems[buf]
            return [
                pltpu.async_copy(edge_hbm.at[pl.ds(eoff, CHUNK)],
                                 src_bufs[buf], sem),
                pltpu.async_copy(edge_hbm.at[pl.ds(E + eoff, CHUNK)],
                                 dst_bufs[buf], sem),
                pltpu.async_copy(w_hbm.at[pl.ds(eoff, CHUNK)],
                                 w_bufs[buf], sem),
            ]

        tcopies = _load_table(table_hbm)
        bcopy = pltpu.async_copy(b1_hbm, b1_v, sem_t)
        _edge_prefetch = _fire(0)
        _zero_acc()
        for t in tcopies:
            t.wait()
        bcopy.wait()
        for ci in range(NCHUNK):
            buf = ci % 2
            for hdl in _edge_prefetch:
                hdl.wait()
            if ci + 1 < NCHUNK:
                _edge_prefetch = _fire(ci + 1)
            src_v, dst_v, w_v = src_bufs[buf], dst_bufs[buf], w_bufs[buf]

            @plsc.parallel_loop(0, CHUNK, 16, unroll=8)
            def _edges1(i):
                base = pl.multiple_of(i, 16)
                srcs = src_v[pl.ds(base, 16)]
                dsts = dst_v[pl.ds(base, 16)]
                wv = w_v[pl.ds(base, 16)]
                for j in range(F):
                    vals = plsc.load_gather(table_v, [srcs + (j * N)])
                    plsc.addupdate_scatter(acc_v, [dsts + (j * N)],
                                           vals * wv)

        for j in range(F):
            pltpu.sync_copy(acc_v.at[pl.ds(j * N, N)],
                            stage_hbm.at[pl.ds((q * H + f0 + j) * N, N)])

        plsc.subcore_barrier()

        o0 = q * SLSTEP
        for j in range(F):
            rcopies = [
                pltpu.async_copy(
                    stage_hbm.at[pl.ds((qq * H + f0 + j) * N + o0, SL)],
                    red_v.at[pl.ds(qq * SL, SL)], sem_t)
                for qq in range(NQ)
            ]
            for r in rcopies:
                r.wait()
            bias = plsc.load_gather(b1_v, [jnp.zeros((16,), jnp.int32)
                                           + (f0 + j)])

            @plsc.parallel_loop(0, SL, 16, unroll=4)
            def _reduce(i):
                off = pl.multiple_of(i, 16)
                acc16 = red_v[pl.ds(off, 16)]
                for qq in range(1, NQ):
                    acc16 = acc16 + red_v[pl.ds(qq * SL + off, 16)]
                comb_v[pl.ds(off, 16)] = jnp.maximum(acc16 + bias, 0.0)

            pltpu.sync_copy(comb_v,
                            rt_hbm.at[pl.ds((f0 + j) * N + o0, SL)])

        plsc.subcore_barrier()

        tcopies = _load_table(rt_hbm)
        _edge_prefetch = _fire(0)
        _zero_acc()
        for t in tcopies:
            t.wait()
        for ci in range(NCHUNK):
            buf = ci % 2
            for hdl in _edge_prefetch:
                hdl.wait()
            if ci + 1 < NCHUNK:
                _edge_prefetch = _fire(ci + 1)
            src_v, dst_v, w_v = src_bufs[buf], dst_bufs[buf], w_bufs[buf]

            @plsc.parallel_loop(0, CHUNK, 16, unroll=8)
            def _edges2(i):
                base = pl.multiple_of(i, 16)
                srcs = src_v[pl.ds(base, 16)]
                dsts = dst_v[pl.ds(base, 16)]
                wv = w_v[pl.ds(base, 16)]
                for j in range(F):
                    vals = plsc.load_gather(table_v, [srcs + (j * N)])
                    plsc.addupdate_scatter(acc_v, [dsts + (j * N)],
                                           vals * wv)

        for j in range(F):
            pltpu.sync_copy(acc_v.at[pl.ds(j * N, N)],
                            p2_hbm.at[pl.ds((q * H + f0 + j) * N, N)])

    return gcn(table_flat, edge_index, w, b1)


def _project_kernel(x_ref, w1_ref, out_ref):
    out_ref[...] = lax.dot_general(
        w1_ref[...], x_ref[...], (((0,), (1,)), ((), ())),
        preferred_element_type=jnp.float32)


def _final_kernel(p_ref, w2_ref, b2_ref, out_ref):
    s = p_ref[0:H, :]
    for q in range(1, NQ):
        s = s + p_ref[q * H:(q + 1) * H, :]
    z = lax.dot_general(s, w2_ref[...], (((0,), (0,)), ((), ())),
                        preferred_element_type=jnp.float32)
    z = z + b2_ref[...]
    m = jnp.max(z, axis=1, keepdims=True)
    zm = z - m
    lse = jnp.log(jnp.sum(jnp.exp(zm), axis=1, keepdims=True))
    out_ref[...] = zm - lse


def kernel(x, edge_index, edge_attr, W1, b1, W2, b2):
    h1_t = pl.pallas_call(
        _project_kernel,
        out_shape=jax.ShapeDtypeStruct((H, N), jnp.float32),
    )(x, W1)

    p2, _, _ = _sc_gcn(h1_t.reshape(H * N), edge_index.reshape(2 * E),
                       edge_attr, b1)

    out = pl.pallas_call(
        _final_kernel,
        out_shape=jax.ShapeDtypeStruct((N, C), jnp.float32),
    )(p2.reshape(NQ * H, N), W2, b2)

    return out


# speedup vs baseline: 28.9650x; 1.0116x over previous
import functools

import jax
import jax.numpy as jnp
from jax import lax
from jax.experimental import pallas as pl
from jax.experimental.pallas import tpu as pltpu
from jax.experimental.pallas import tpu_sc as plsc

N = 10000
E = 320000
D = 128
H = 16
C = 40

NQ = 8
F = 4
EPQ = E // NQ
CHUNK = 4000
NCHUNK = EPQ // CHUNK
SL = 1264
SLSTEP = 1248


def _sc_gcn(table_flat, edge_index, w, b1):
    mesh = plsc.VectorSubcoreMesh(core_axis_name="c", subcore_axis_name="s")

    @functools.partial(
        pl.kernel,
        mesh=mesh,
        compiler_params=pltpu.CompilerParams(needs_layout_passes=False),
        out_type=[
            jax.ShapeDtypeStruct((H * N,), jnp.float32),
            jax.ShapeDtypeStruct((NQ * H * N,), jnp.float32),
            jax.ShapeDtypeStruct((H * N,), jnp.float32),
        ],
        scratch_types=[
            pltpu.VMEM((F * N,), jnp.float32),
            pltpu.VMEM((F * N,), jnp.float32),
            pltpu.VMEM((CHUNK,), jnp.int32),
            pltpu.VMEM((CHUNK,), jnp.int32),
            pltpu.VMEM((CHUNK,), jnp.int32),
            pltpu.VMEM((CHUNK,), jnp.int32),
            pltpu.VMEM((CHUNK,), jnp.float32),
            pltpu.VMEM((CHUNK,), jnp.float32),
            pltpu.VMEM((NQ * SL,), jnp.float32),
            pltpu.VMEM((NQ * SL,), jnp.float32),
            pltpu.VMEM((SL,), jnp.float32),
            pltpu.VMEM((16,), jnp.float32),
            pltpu.SemaphoreType.DMA,
            pltpu.SemaphoreType.DMA,
            pltpu.SemaphoreType.DMA,
        ],
    )
    def gcn(table_hbm, edge_hbm, w_hbm, b1_hbm, r2_hbm, stage_hbm, rt_hbm,
            table_v, acc_v, src_v0, src_v1, dst_v0, dst_v1, w_v0, w_v1,
            red_v0, red_v1, comb_v, b1_v, sem_t, sem_c0, sem_c1):
        c = lax.axis_index("c")
        s = lax.axis_index("s")
        g = s // 8
        q = s % 8
        f0 = 8 * c + F * g
        e0 = q * EPQ
        o0 = q * SLSTEP
        sems = [sem_c0, sem_c1]
        src_bufs = [src_v0, src_v1]
        dst_bufs = [dst_v0, dst_v1]
        w_bufs = [w_v0, w_v1]
        red_bufs = [red_v0, red_v1]

        def _load_table(src_hbm_flat):
            return [
                pltpu.async_copy(src_hbm_flat.at[pl.ds((f0 + j) * N, N)],
                                 table_v.at[pl.ds(j * N, N)], sem_t)
                for j in range(F)
            ]

        def _zero_acc():
            @plsc.parallel_loop(0, F * N, 16, unroll=8)
            def _zero(i):
                off = pl.multiple_of(i, 16)
                acc_v[pl.ds(off, 16)] = jnp.zeros((16,), jnp.float32)

        def _fire(ci):
            buf = ci % 2
            eoff = e0 + ci * CHUNK
            sem = sems[buf]
            return [
                pltpu.async_copy(edge_hbm.at[pl.ds(eoff, CHUNK)],
                                 src_bufs[buf], sem),
                pltpu.async_copy(edge_hbm.at[pl.ds(E + eoff, CHUNK)],
                                 dst_bufs[buf], sem),
                pltpu.async_copy(w_hbm.at[pl.ds(eoff, CHUNK)],
                                 w_bufs[buf], sem),
            ]

        def _edge_loop(prefetch):
            for ci in range(NCHUNK):
                buf = ci % 2
                for hdl in prefetch:
                    hdl.wait()
                if ci + 1 < NCHUNK:
                    prefetch = _fire(ci + 1)
                src_v, dst_v, w_v = (src_bufs[buf], dst_bufs[buf],
                                     w_bufs[buf])

                @plsc.parallel_loop(0, CHUNK, 16, unroll=8)
                def _edges(i):
                    base = pl.multiple_of(i, 16)
                    srcs = src_v[pl.ds(base, 16)]
                    dsts = dst_v[pl.ds(base, 16)]
                    wv = w_v[pl.ds(base, 16)]
                    for j in range(F):
                        vals = plsc.load_gather(table_v, [srcs + (j * N)])
                        plsc.addupdate_scatter(acc_v, [dsts + (j * N)],
                                               vals * wv)

        def _stage_partials():
            hs = [
                pltpu.async_copy(acc_v.at[pl.ds(j * N, N)],
                                 stage_hbm.at[pl.ds((q * H + f0 + j) * N, N)],
                                 sem_t)
                for j in range(F)
            ]
            for hdl in hs:
                hdl.wait()

        def _fire_slices(j):
            sem = sems[j % 2]
            return [
                pltpu.async_copy(
                    stage_hbm.at[pl.ds((qq * H + f0 + j) * N + o0, SL)],
                    red_bufs[j % 2].at[pl.ds(qq * SL, SL)], sem)
                for qq in range(NQ)
            ]

        def _combine(dst_flat_hbm, with_bias_relu):
            prefetch = _fire_slices(0)
            for j in range(F):
                red_v = red_bufs[j % 2]
                for hdl in prefetch:
                    hdl.wait()
                if j + 1 < F:
                    prefetch = _fire_slices(j + 1)
                if with_bias_relu:
                    bias = plsc.load_gather(
                        b1_v, [jnp.zeros((16,), jnp.int32) + (f0 + j)])

                @plsc.parallel_loop(0, SL, 16, unroll=4)
                def _reduce(i):
                    off = pl.multiple_of(i, 16)
                    acc16 = red_v[pl.ds(off, 16)]
                    for qq in range(1, NQ):
                        acc16 = acc16 + red_v[pl.ds(qq * SL + off, 16)]
                    if with_bias_relu:
                        acc16 = jnp.maximum(acc16 + bias, 0.0)
                    comb_v[pl.ds(off, 16)] = acc16

                pltpu.sync_copy(comb_v,
                                dst_flat_hbm.at[pl.ds((f0 + j) * N + o0, SL)])

        tcopies = _load_table(table_hbm)
        bcopy = pltpu.async_copy(b1_hbm, b1_v, sem_t)
        prefetch = _fire(0)
        _zero_acc()
        for t in tcopies:
            t.wait()
        bcopy.wait()
        _edge_loop(prefetch)
        _stage_partials()
        plsc.subcore_barrier()
        _combine(rt_hbm, with_bias_relu=True)
        plsc.subcore_barrier()

        tcopies = _load_table(rt_hbm)
        prefetch = _fire(0)
        _zero_acc()
        for t in tcopies:
            t.wait()
        _edge_loop(prefetch)
        _stage_partials()
        plsc.subcore_barrier()
        _combine(r2_hbm, with_bias_relu=False)

    return gcn(table_flat, edge_index, w, b1)


def _project_kernel(x_ref, w1_ref, out_ref):
    out_ref[...] = lax.dot_general(
        w1_ref[...], x_ref[...], (((0,), (1,)), ((), ())),
        preferred_element_type=jnp.float32)


def _final_kernel(r2_ref, w2_ref, b2_ref, out_ref):
    z = lax.dot_general(r2_ref[...], w2_ref[...], (((0,), (0,)), ((), ())),
                        preferred_element_type=jnp.float32)
    z = z + b2_ref[...]
    m = jnp.max(z, axis=1, keepdims=True)
    zm = z - m
    lse = jnp.log(jnp.sum(jnp.exp(zm), axis=1, keepdims=True))
    out_ref[...] = zm - lse


def kernel(x, edge_index, edge_attr, W1, b1, W2, b2):
    h1_t = pl.pallas_call(
        _project_kernel,
        out_shape=jax.ShapeDtypeStruct((H, N), jnp.float32),
    )(x, W1)

    r2t, _, _ = _sc_gcn(h1_t.reshape(H * N), edge_index.reshape(2 * E),
                        edge_attr, b1)

    out = pl.pallas_call(
        _final_kernel,
        out_shape=jax.ShapeDtypeStruct((N, C), jnp.float32),
    )(r2t.reshape(H, N), W2, b2)

    return out


# speedup vs baseline: 31.0837x; 1.0731x over previous
import functools

import jax
import jax.numpy as jnp
from jax import lax
from jax.experimental import pallas as pl
from jax.experimental.pallas import tpu as pltpu
from jax.experimental.pallas import tpu_sc as plsc

N = 10000
E = 320000
D = 128
H = 16
C = 40

NQ = 8
F = 4
EPQ = E // NQ
CHUNK = 4000
NCHUNK = EPQ // CHUNK
SL = 1264
SLSTEP = 1248


def _sc_gcn(table_flat, edge_index, w, b1):
    mesh = plsc.VectorSubcoreMesh(core_axis_name="c", subcore_axis_name="s")

    @functools.partial(
        pl.kernel,
        mesh=mesh,
        compiler_params=pltpu.CompilerParams(needs_layout_passes=False),
        out_type=[
            jax.ShapeDtypeStruct((H * N,), jnp.float32),
            jax.ShapeDtypeStruct((NQ * H * N,), jnp.float32),
            jax.ShapeDtypeStruct((H * N,), jnp.float32),
        ],
        scratch_types=[
            pltpu.VMEM((F * N,), jnp.float32),
            pltpu.VMEM((F * N,), jnp.float32),
            pltpu.VMEM((CHUNK,), jnp.int32),
            pltpu.VMEM((CHUNK,), jnp.int32),
            pltpu.VMEM((CHUNK,), jnp.int32),
            pltpu.VMEM((CHUNK,), jnp.int32),
            pltpu.VMEM((CHUNK,), jnp.float32),
            pltpu.VMEM((CHUNK,), jnp.float32),
            pltpu.VMEM((NQ * SL,), jnp.float32),
            pltpu.VMEM((NQ * SL,), jnp.float32),
            pltpu.VMEM((SL,), jnp.float32),
            pltpu.VMEM((16,), jnp.float32),
            pltpu.SemaphoreType.DMA,
            pltpu.SemaphoreType.DMA,
            pltpu.SemaphoreType.DMA,
        ],
    )
    def gcn(table_hbm, edge_hbm, w_hbm, b1_hbm, r2_hbm, stage_hbm, rt_hbm,
            table_v, acc_v, src_v0, src_v1, dst_v0, dst_v1, w_v0, w_v1,
            red_v0, red_v1, comb_v, b1_v, sem_t, sem_c0, sem_c1):
        c = lax.axis_index("c")
        s = lax.axis_index("s")
        g = s // 8
        q = s % 8
        f0 = 8 * c + F * g
        e0 = q * EPQ
        o0 = q * SLSTEP
        sems = [sem_c0, sem_c1]
        src_bufs = [src_v0, src_v1]
        dst_bufs = [dst_v0, dst_v1]
        w_bufs = [w_v0, w_v1]
        red_bufs = [red_v0, red_v1]

        def _load_table(src_hbm_flat):
            return [
                pltpu.async_copy(src_hbm_flat.at[pl.ds((f0 + j) * N, N)],
                                 table_v.at[pl.ds(j * N, N)], sem_t)
                for j in range(F)
            ]

        def _zero_acc():
            @plsc.parallel_loop(0, F * N, 16, unroll=8)
            def _zero(i):
                off = pl.multiple_of(i, 16)
                acc_v[pl.ds(off, 16)] = jnp.zeros((16,), jnp.float32)

        def _fire(ci):
            buf = ci % 2
            eoff = e0 + ci * CHUNK
            sem = sems[buf]
            return [
                pltpu.async_copy(edge_hbm.at[pl.ds(eoff, CHUNK)],
                                 src_bufs[buf], sem),
                pltpu.async_copy(edge_hbm.at[pl.ds(E + eoff, CHUNK)],
                                 dst_bufs[buf], sem),
                pltpu.async_copy(w_hbm.at[pl.ds(eoff, CHUNK)],
                                 w_bufs[buf], sem),
            ]

        def _edge_loop(prefetch):
            for ci in range(NCHUNK):
                buf = ci % 2
                for hdl in prefetch:
                    hdl.wait()
                if ci + 1 < NCHUNK:
                    prefetch = _fire(ci + 1)
                src_v, dst_v, w_v = (src_bufs[buf], dst_bufs[buf],
                                     w_bufs[buf])

                @plsc.parallel_loop(0, CHUNK, 16, unroll=8)
                def _edges(i):
                    base = pl.multiple_of(i, 16)
                    srcs = src_v[pl.ds(base, 16)]
                    dsts = dst_v[pl.ds(base, 16)]
                    wv = w_v[pl.ds(base, 16)]
                    for j in range(F):
                        vals = plsc.load_gather(table_v, [srcs + (j * N)])
                        plsc.addupdate_scatter(acc_v, [dsts + (j * N)],
                                               vals * wv)

        def _stage_partials():
            hs = [
                pltpu.async_copy(acc_v.at[pl.ds(j * N, N)],
                                 stage_hbm.at[pl.ds((q * H + f0 + j) * N, N)],
                                 sem_t)
                for j in range(F)
            ]
            for hdl in hs:
                hdl.wait()

        def _fire_slices(j):
            sem = sems[j % 2]
            return [
                pltpu.async_copy(
                    stage_hbm.at[pl.ds((qq * H + f0 + j) * N + o0, SL)],
                    red_bufs[j % 2].at[pl.ds(qq * SL, SL)], sem)
                for qq in range(NQ)
            ]

        def _combine(dst_flat_hbm, with_bias_relu):
            prefetch = _fire_slices(0)
            for j in range(F):
                red_v = red_bufs[j % 2]
                for hdl in prefetch:
                    hdl.wait()
                if j + 1 < F:
                    prefetch = _fire_slices(j + 1)
                if with_bias_relu:
                    bias = plsc.load_gather(
                        b1_v, [jnp.zeros((16,), jnp.int32) + (f0 + j)])

                @plsc.parallel_loop(0, SL, 16, unroll=4)
                def _reduce(i):
                    off = pl.multiple_of(i, 16)
                    acc16 = red_v[pl.ds(off, 16)]
                    for qq in range(1, NQ):
                        acc16 = acc16 + red_v[pl.ds(qq * SL + off, 16)]
                    if with_bias_relu:
                        acc16 = jnp.maximum(acc16 + bias, 0.0)
                    comb_v[pl.ds(off, 16)] = acc16

                pltpu.sync_copy(comb_v,
                                dst_flat_hbm.at[pl.ds((f0 + j) * N + o0, SL)])

        tcopies = _load_table(table_hbm)
        bcopy = pltpu.async_copy(b1_hbm, b1_v, sem_t)
        prefetch = _fire(0)
        _zero_acc()
        for t in tcopies:
            t.wait()
        bcopy.wait()
        _edge_loop(prefetch)
        _stage_partials()
        plsc.subcore_barrier()
        _combine(rt_hbm, with_bias_relu=True)
        plsc.subcore_barrier()

        tcopies = _load_table(rt_hbm)
        prefetch = _fire(0)
        _zero_acc()
        for t in tcopies:
            t.wait()
        _edge_loop(prefetch)
        _stage_partials()
        plsc.subcore_barrier()
        _combine(r2_hbm, with_bias_relu=False)

    return gcn(table_flat, edge_index, w, b1)


def _project_kernel(x_ref, w1_ref, out_ref):
    out_ref[...] = lax.dot_general(
        w1_ref[...], x_ref[...], (((0,), (1,)), ((), ())),
        preferred_element_type=jnp.float32)


def _final_kernel(r2_ref, w2_ref, b2_ref, out_ref):
    z = lax.dot_general(w2_ref[...], r2_ref[...], (((0,), (0,)), ((), ())),
                        preferred_element_type=jnp.float32)
    z = z + b2_ref[...]
    m = jnp.max(z, axis=0, keepdims=True)
    zm = z - m
    lse = jnp.log(jnp.sum(jnp.exp(zm), axis=0, keepdims=True))
    out_ref[...] = zm - lse


def kernel(x, edge_index, edge_attr, W1, b1, W2, b2):
    h1_t = pl.pallas_call(
        _project_kernel,
        out_shape=jax.ShapeDtypeStruct((H, N), jnp.float32),
    )(x, W1)

    r2t, _, _ = _sc_gcn(h1_t.reshape(H * N), edge_index.reshape(2 * E),
                        edge_attr, b1)

    out_t = pl.pallas_call(
        _final_kernel,
        out_shape=jax.ShapeDtypeStruct((C, N), jnp.float32),
    )(r2t.reshape(H, N), W2, b2.reshape(C, 1))

    return out_t.T


# speedup vs baseline: 35.2277x; 1.1333x over previous
import functools

import jax
import jax.numpy as jnp
from jax import lax
from jax.experimental import pallas as pl
from jax.experimental.pallas import tpu as pltpu
from jax.experimental.pallas import tpu_sc as plsc

N = 10000
E = 320000
D = 128
H = 16
C = 40

NQ = 8
F = 4
FP = F // 2
EPQ = E // NQ
CHUNK = 4000
NCHUNK = EPQ // CHUNK
SL = 1264
SLSTEP = 1248


def _sc_gcn(ptable_flat, edge_index, w, b1):
    mesh = plsc.VectorSubcoreMesh(core_axis_name="c", subcore_axis_name="s")

    @functools.partial(
        pl.kernel,
        mesh=mesh,
        compiler_params=pltpu.CompilerParams(needs_layout_passes=False),
        out_type=[
            jax.ShapeDtypeStruct((H * N,), jnp.float32),
            jax.ShapeDtypeStruct((NQ * H * N,), jnp.float32),
            jax.ShapeDtypeStruct((H // 2 * N,), jnp.int32),
        ],
        scratch_types=[
            pltpu.VMEM((FP * N,), jnp.int32),
            pltpu.VMEM((F * N,), jnp.float32),
            pltpu.VMEM((CHUNK,), jnp.int32),
            pltpu.VMEM((CHUNK,), jnp.int32),
            pltpu.VMEM((CHUNK,), jnp.int32),
            pltpu.VMEM((CHUNK,), jnp.int32),
            pltpu.VMEM((CHUNK,), jnp.float32),
            pltpu.VMEM((CHUNK,), jnp.float32),
            pltpu.VMEM((NQ * SL,), jnp.float32),
            pltpu.VMEM((NQ * SL,), jnp.float32),
            pltpu.VMEM((NQ * SL,), jnp.float32),
            pltpu.VMEM((NQ * SL,), jnp.float32),
            pltpu.VMEM((SL,), jnp.int32),
            pltpu.VMEM((SL,), jnp.float32),
            pltpu.VMEM((16,), jnp.float32),
            pltpu.SemaphoreType.DMA,
            pltpu.SemaphoreType.DMA,
            pltpu.SemaphoreType.DMA,
        ],
    )
    def gcn(ptable_hbm, edge_hbm, w_hbm, b1_hbm, r2_hbm, stage_hbm, prt_hbm,
            table_v, acc_v, src_v0, src_v1, dst_v0, dst_v1, w_v0, w_v1,
            red_a0, red_b0, red_a1, red_b1, combp_v, combf_v, b1_v,
            sem_t, sem_c0, sem_c1):
        c = lax.axis_index("c")
        s = lax.axis_index("s")
        g = s // 8
        q = s % 8
        f0 = 8 * c + F * g
        p0 = f0 // 2
        e0 = q * EPQ
        o0 = q * SLSTEP
        sems = [sem_c0, sem_c1]
        src_bufs = [src_v0, src_v1]
        dst_bufs = [dst_v0, dst_v1]
        w_bufs = [w_v0, w_v1]
        red_a = [red_a0, red_a1]
        red_b = [red_b0, red_b1]

        def _load_table(src_hbm_packed):
            return [
                pltpu.async_copy(src_hbm_packed.at[pl.ds((p0 + p) * N, N)],
                                 table_v.at[pl.ds(p * N, N)], sem_t)
                for p in range(FP)
            ]

        def _zero_acc():
            @plsc.parallel_loop(0, F * N, 16, unroll=8)
            def _zero(i):
                off = pl.multiple_of(i, 16)
                acc_v[pl.ds(off, 16)] = jnp.zeros((16,), jnp.float32)

        def _fire(ci):
            buf = ci % 2
            eoff = e0 + ci * CHUNK
            sem = sems[buf]
            return [
                pltpu.async_copy(edge_hbm.at[pl.ds(eoff, CHUNK)],
                                 src_bufs[buf], sem),
                pltpu.async_copy(edge_hbm.at[pl.ds(E + eoff, CHUNK)],
                                 dst_bufs[buf], sem),
                pltpu.async_copy(w_hbm.at[pl.ds(eoff, CHUNK)],
                                 w_bufs[buf], sem),
            ]

        def _edge_loop(prefetch):
            for ci in range(NCHUNK):
                buf = ci % 2
                for hdl in prefetch:
                    hdl.wait()
                if ci + 1 < NCHUNK:
                    prefetch = _fire(ci + 1)
                src_v, dst_v, w_v = (src_bufs[buf], dst_bufs[buf],
                                     w_bufs[buf])

                @plsc.parallel_loop(0, CHUNK, 16, unroll=8)
                def _edges(i):
                    base = pl.multiple_of(i, 16)
                    srcs = src_v[pl.ds(base, 16)]
                    dsts = dst_v[pl.ds(base, 16)]
                    wv = w_v[pl.ds(base, 16)]
                    for p in range(FP):
                        pk = plsc.load_gather(table_v, [srcs + (p * N)])
                        va, vb = plsc.unpack(
                            plsc.bitcast(pk, jnp.bfloat16),
                            format=plsc.PackFormat.INTERLEAVED)
                        plsc.addupdate_scatter(
                            acc_v, [dsts + ((2 * p) * N)], va * wv)
                        plsc.addupdate_scatter(
                            acc_v, [dsts + ((2 * p + 1) * N)], vb * wv)

        def _stage_partials():
            hs = [
                pltpu.async_copy(acc_v.at[pl.ds(j * N, N)],
                                 stage_hbm.at[pl.ds((q * H + f0 + j) * N, N)],
                                 sem_t)
                for j in range(F)
            ]
            for hdl in hs:
                hdl.wait()

        def _fire_slices(p):
            sem = sems[p % 2]
            hs = []
            for qq in range(NQ):
                hs.append(pltpu.async_copy(
                    stage_hbm.at[pl.ds((qq * H + f0 + 2 * p) * N + o0, SL)],
                    red_a[p % 2].at[pl.ds(qq * SL, SL)], sem))
                hs.append(pltpu.async_copy(
                    stage_hbm.at[pl.ds((qq * H + f0 + 2 * p + 1) * N + o0,
                                       SL)],
                    red_b[p % 2].at[pl.ds(qq * SL, SL)], sem))
            return hs

        def _combine_pack_relu():
            prefetch = _fire_slices(0)
            for p in range(FP):
                ra, rb = red_a[p % 2], red_b[p % 2]
                for hdl in prefetch:
                    hdl.wait()
                if p + 1 < FP:
                    prefetch = _fire_slices(p + 1)
                zero16 = jnp.zeros((16,), jnp.int32)
                bias_a = plsc.load_gather(b1_v, [zero16 + (f0 + 2 * p)])
                bias_b = plsc.load_gather(b1_v, [zero16 + (f0 + 2 * p + 1)])

                @plsc.parallel_loop(0, SL, 16, unroll=4)
                def _reduce(i):
                    off = pl.multiple_of(i, 16)
                    acc_a = ra[pl.ds(off, 16)]
                    acc_b = rb[pl.ds(off, 16)]
                    for qq in range(1, NQ):
                        acc_a = acc_a + ra[pl.ds(qq * SL + off, 16)]
                        acc_b = acc_b + rb[pl.ds(qq * SL + off, 16)]
                    acc_a = jnp.maximum(acc_a + bias_a, 0.0)
                    acc_b = jnp.maximum(acc_b + bias_b, 0.0)
                    pk = plsc.pack(acc_a, acc_b,
                                   format=plsc.PackFormat.INTERLEAVED)
                    combp_v[pl.ds(off, 16)] = plsc.bitcast(pk, jnp.int32)

                pltpu.sync_copy(combp_v,
                                prt_hbm.at[pl.ds((p0 + p) * N + o0, SL)])

        def _combine_f32():
            prefetch = _fire_slices(0)
            for p in range(FP):
                ra, rb = red_a[p % 2], red_b[p % 2]
                for hdl in prefetch:
                    hdl.wait()
                if p + 1 < FP:
                    prefetch = _fire_slices(p + 1)
                for which, red_v in ((0, ra), (1, rb)):
                    @plsc.parallel_loop(0, SL, 16, unroll=4)
                    def _reduce(i):
                        off = pl.multiple_of(i, 16)
                        acc16 = red_v[pl.ds(off, 16)]
                        for qq in range(1, NQ):
                            acc16 = acc16 + red_v[pl.ds(qq * SL + off, 16)]
                        combf_v[pl.ds(off, 16)] = acc16

                    pltpu.sync_copy(
                        combf_v,
                        r2_hbm.at[pl.ds((f0 + 2 * p + which) * N + o0, SL)])

        tcopies = _load_table(ptable_hbm)
        bcopy = pltpu.async_copy(b1_hbm, b1_v, sem_t)
        prefetch = _fire(0)
        _zero_acc()
        for t in tcopies:
            t.wait()
        bcopy.wait()
        _edge_loop(prefetch)
        _stage_partials()
        plsc.subcore_barrier()
        _combine_pack_relu()
        plsc.subcore_barrier()

        tcopies = _load_table(prt_hbm)
        prefetch = _fire(0)
        _zero_acc()
        for t in tcopies:
            t.wait()
        _edge_loop(prefetch)
        _stage_partials()
        plsc.subcore_barrier()
        _combine_f32()

    return gcn(ptable_flat, edge_index, w, b1)


def _project_kernel(x_ref, w1_ref, out_ref):
    h = lax.dot_general(w1_ref[...], x_ref[...], (((0,), (1,)), ((), ())),
                        preferred_element_type=jnp.float32)
    h3 = h.reshape(H // 2, 2, N)
    he = lax.bitcast_convert_type(h3[:, 0, :].astype(jnp.bfloat16),
                                  jnp.uint16).astype(jnp.int32)
    ho = lax.bitcast_convert_type(h3[:, 1, :].astype(jnp.bfloat16),
                                  jnp.uint16).astype(jnp.int32)
    out_ref[...] = he | (ho << 16)


def _final_kernel(r2_ref, w2_ref, b2_ref, out_ref):
    z = lax.dot_general(w2_ref[...], r2_ref[...], (((0,), (0,)), ((), ())),
                        preferred_element_type=jnp.float32)
    z = z + b2_ref[...]
    m = jnp.max(z, axis=0, keepdims=True)
    zm = z - m
    lse = jnp.log(jnp.sum(jnp.exp(zm), axis=0, keepdims=True))
    out_ref[...] = zm - lse


def kernel(x, edge_index, edge_attr, W1, b1, W2, b2):
    h1_pk = pl.pallas_call(
        _project_kernel,
        out_shape=jax.ShapeDtypeStruct((H // 2, N), jnp.int32),
    )(x, W1)

    r2t, _, _ = _sc_gcn(h1_pk.reshape(H // 2 * N), edge_index.reshape(2 * E),
                        edge_attr, b1)

    out_t = pl.pallas_call(
        _final_kernel,
        out_shape=jax.ShapeDtypeStruct((C, N), jnp.float32),
    )(r2t.reshape(H, N), W2, b2.reshape(C, 1))

    return out_t.T
